# Initial kernel scaffold; baseline (speedup 1.0000x reference)
import jax
import jax.numpy as jnp
from jax.experimental import pallas as pl


def kernel(input):
    raise NotImplementedError("write your pallas kernel here")


# speedup vs baseline: 6.2987x; 6.2987x over previous
import jax
import jax.numpy as jnp
from jax import lax
from jax.experimental import pallas as pl

KMAX = 10
KMIN = 10
ROWS_PER_BLOCK = 256


def _body(x_ref, o_ref):
    x = x_ref[...]
    R, N = x.shape
    col = lax.broadcasted_iota(jnp.int32, (R, N), 1)
    neg = jnp.float32(-jnp.inf)
    pos = jnp.float32(jnp.inf)

    xt = x
    top_sum = jnp.zeros((R, 1), jnp.float32)
    for _ in range(KMAX):
        m = jnp.max(xt, axis=1, keepdims=True)
        top_sum = top_sum + m
        idx = jnp.min(jnp.where(xt == m, col, N), axis=1, keepdims=True)
        xt = jnp.where(col == idx, neg, xt)

    xb = x
    bot_sum = jnp.zeros((R, 1), jnp.float32)
    for _ in range(KMIN):
        m = jnp.min(xb, axis=1, keepdims=True)
        bot_sum = bot_sum + m
        idx = jnp.min(jnp.where(xb == m, col, N), axis=1, keepdims=True)
        xb = jnp.where(col == idx, pos, xb)

    o_ref[...] = (top_sum / KMAX + bot_sum / KMIN) * 0.5


def kernel(input):
    B, C, H, W = input.shape
    n = H * W
    rows = B * C
    x = input.reshape(rows, n)
    R = ROWS_PER_BLOCK
    grid = (rows // R,)
    out = pl.pallas_call(
        _body,
        grid=grid,
        in_specs=[pl.BlockSpec((R, n), lambda i: (i, 0))],
        out_specs=pl.BlockSpec((R, 1), lambda i: (i, 0)),
        out_shape=jax.ShapeDtypeStruct((rows, 1), jnp.float32),
    )(x)
    return out.reshape(B, C)
